# baseline (speedup 1.0000x reference)
import functools

import jax
import jax.numpy as jnp
from jax import lax
from jax.experimental import pallas as pl
from jax.experimental.pallas import tpu as pltpu
from jax.experimental.pallas import tpu_sc as plsc

VOCAB = 100000
MAXLEN = 200
EMBED = 128
BATCH = 1024

NC = 2
NS = 16
L = 16
NW = NC * NS
C = 128
CHUNKS = BATCH // C
PRANGES = NW // CHUNKS
PPW = MAXLEN // PRANGES
NBUF = 4


@jax.jit
def _embed(x_flat, token_table, pos_table):
    mesh = plsc.VectorSubcoreMesh(
        core_axis_name="c", subcore_axis_name="s", num_cores=NC, num_subcores=NS
    )

    @functools.partial(
        pl.kernel,
        mesh=mesh,
        compiler_params=pltpu.CompilerParams(
            use_tc_tiling_on_sc=False, needs_layout_passes=False
        ),
        out_type=jax.ShapeDtypeStruct((BATCH, MAXLEN * EMBED), jnp.float32),
        scratch_types=[
            pltpu.VMEM((C * MAXLEN,), jnp.int32),
            pltpu.VMEM((NBUF, C), jnp.int32),
            pltpu.VMEM((C, EMBED), jnp.float32),
            pltpu.VMEM((C, EMBED), jnp.float32),
            pltpu.VMEM((C, EMBED), jnp.float32),
            pltpu.VMEM((C, EMBED), jnp.float32),
            pltpu.VMEM((PPW, EMBED), jnp.float32),
            pltpu.SemaphoreType.DMA,
            pltpu.SemaphoreType.DMA,
            pltpu.SemaphoreType.DMA,
            pltpu.SemaphoreType.DMA,
            pltpu.SemaphoreType.DMA,
            pltpu.SemaphoreType.DMA,
            pltpu.SemaphoreType.DMA,
            pltpu.SemaphoreType.DMA,
        ],
    )
    def k(x_hbm, table_hbm, pos_hbm, out_hbm,
          xblk, idx, rows0, rows1, rows2, rows3, pos_vm,
          gsem0, gsem1, gsem2, gsem3, wsem0, wsem1, wsem2, wsem3):
        rows = [rows0, rows1, rows2, rows3]
        gsem = [gsem0, gsem1, gsem2, gsem3]
        wsem = [wsem0, wsem1, wsem2, wsem3]

        wid = lax.axis_index("s") * NC + lax.axis_index("c")
        ch = wid % CHUNKS
        q = wid // CHUNKS
        b0 = ch * C
        p0 = q * PPW

        pltpu.sync_copy(x_hbm.at[pl.ds(b0 * MAXLEN, C * MAXLEN)], xblk)

        iota = lax.iota(jnp.int32, L)
        row_base = iota * MAXLEN

        def build_idx(u, b):
            col = row_base + (p0 + u)

            def body(j, carry):
                vals = plsc.load_gather(xblk, [col + j * (L * MAXLEN)])
                idx[b, pl.ds(j * L, L)] = vals
                return carry

            lax.fori_loop(0, C // L, body, 0)

        def start_gather(b):
            return pltpu.async_copy(table_hbm.at[idx.at[b]], rows[b], gsem[b])

        pending_g = [None] * NBUF
        pending_w = [None] * NBUF
        for j in range(NBUF - 1):
            build_idx(j, j)
            pending_g[j] = start_gather(j)
        pltpu.sync_copy(pos_hbm.at[pl.ds(p0, PPW)], pos_vm)

        for u in range(PPW):
            b = u % NBUF
            if u + NBUF - 1 < PPW:
                nb = (u + NBUF - 1) % NBUF
                build_idx(u + NBUF - 1, nb)
                if pending_w[nb] is not None:
                    pending_w[nb].wait()
                    pending_w[nb] = None
                pending_g[nb] = start_gather(nb)
            pending_g[b].wait()

            rv = rows[b]
            pvals = [pos_vm[u, pl.ds(c * L, L)] for c in range(EMBED // L)]

            @plsc.parallel_loop(0, C, step=1, unroll=4)
            def add_body(r, rv=rv, pvals=pvals):
                for c in range(EMBED // L):
                    sl = pl.ds(c * L, L)
                    rv[r, sl] = rv[r, sl] + pvals[c]

            pending_w[b] = pltpu.async_copy(
                rv,
                out_hbm.at[pl.ds(b0, C), pl.ds((p0 + u) * EMBED, EMBED)],
                wsem[b],
            )

        for b in range(NBUF):
            if pending_w[b] is not None:
                pending_w[b].wait()

    return k(x_flat, token_table, pos_table)


def kernel(x, token_table, pos_table):
    out = _embed(x.astype(jnp.int32).reshape(-1), token_table, pos_table)
    return out.reshape(BATCH, MAXLEN, EMBED)


# speedup vs baseline: 1.0062x; 1.0062x over previous
import functools

import jax
import jax.numpy as jnp
from jax import lax
from jax.experimental import pallas as pl
from jax.experimental.pallas import tpu as pltpu
from jax.experimental.pallas import tpu_sc as plsc

VOCAB = 100000
MAXLEN = 200
EMBED = 128
BATCH = 1024

NC = 2
NS = 16
L = 16
NW = NC * NS
C = 128
CHUNKS = BATCH // C
PRANGES = NW // CHUNKS
PPW = MAXLEN // PRANGES
NBUF = 4


@jax.jit
def _embed(x_flat, token_table, pos_table):
    mesh = plsc.VectorSubcoreMesh(
        core_axis_name="c", subcore_axis_name="s", num_cores=NC, num_subcores=NS
    )

    @functools.partial(
        pl.kernel,
        mesh=mesh,
        compiler_params=pltpu.CompilerParams(
            use_tc_tiling_on_sc=False, needs_layout_passes=False
        ),
        out_type=jax.ShapeDtypeStruct((BATCH, MAXLEN * EMBED), jnp.float32),
        scratch_types=[
            pltpu.VMEM((C * MAXLEN,), jnp.int32),
            pltpu.VMEM((NBUF, C), jnp.int32),
            pltpu.VMEM((C, EMBED), jnp.float32),
            pltpu.VMEM((C, EMBED), jnp.float32),
            pltpu.VMEM((C, EMBED), jnp.float32),
            pltpu.VMEM((C, EMBED), jnp.float32),
            pltpu.VMEM((PPW, EMBED), jnp.float32),
            pltpu.SemaphoreType.DMA,
            pltpu.SemaphoreType.DMA,
            pltpu.SemaphoreType.DMA,
            pltpu.SemaphoreType.DMA,
            pltpu.SemaphoreType.DMA,
            pltpu.SemaphoreType.DMA,
            pltpu.SemaphoreType.DMA,
            pltpu.SemaphoreType.DMA,
        ],
    )
    def k(x_hbm, table_hbm, pos_hbm, out_hbm,
          xblk, idx, rows0, rows1, rows2, rows3, pos_vm,
          gsem0, gsem1, gsem2, gsem3, wsem0, wsem1, wsem2, wsem3):
        rows = [rows0, rows1, rows2, rows3]
        gsem = [gsem0, gsem1, gsem2, gsem3]
        wsem = [wsem0, wsem1, wsem2, wsem3]

        wid = lax.axis_index("s") * NC + lax.axis_index("c")
        ch = wid % CHUNKS
        q = wid // CHUNKS
        b0 = ch * C
        p0 = q * PPW

        pltpu.sync_copy(x_hbm.at[pl.ds(b0 * MAXLEN, C * MAXLEN)], xblk)

        iota = lax.iota(jnp.int32, L)
        row_base = iota * MAXLEN

        def build_idx(u, b):
            col = row_base + (p0 + u)

            @plsc.parallel_loop(0, C // L, step=1, unroll=4)
            def body(j, col=col, b=b):
                vals = plsc.load_gather(xblk, [col + j * (L * MAXLEN)])
                idx[b, pl.ds(j * L, L)] = vals

        def start_gather(b):
            return pltpu.async_copy(table_hbm.at[idx.at[b]], rows[b], gsem[b])

        pending_g = [None] * NBUF
        pending_w = [None] * NBUF
        for j in range(NBUF - 1):
            build_idx(j, j)
            pending_g[j] = start_gather(j)
        pltpu.sync_copy(pos_hbm.at[pl.ds(p0, PPW)], pos_vm)

        for u in range(PPW):
            b = u % NBUF
            if u + NBUF - 1 < PPW:
                nb = (u + NBUF - 1) % NBUF
                build_idx(u + NBUF - 1, nb)
                if pending_w[nb] is not None:
                    pending_w[nb].wait()
                    pending_w[nb] = None
                pending_g[nb] = start_gather(nb)
            pending_g[b].wait()

            rv = rows[b]
            pvals = [pos_vm[u, pl.ds(c * L, L)] for c in range(EMBED // L)]

            @plsc.parallel_loop(0, C, step=1, unroll=4)
            def add_body(r, rv=rv, pvals=pvals):
                for c in range(EMBED // L):
                    sl = pl.ds(c * L, L)
                    rv[r, sl] = rv[r, sl] + pvals[c]

            pending_w[b] = pltpu.async_copy(
                rv,
                out_hbm.at[pl.ds(b0, C), pl.ds((p0 + u) * EMBED, EMBED)],
                wsem[b],
            )

        for b in range(NBUF):
            if pending_w[b] is not None:
                pending_w[b].wait()

    return k(x_flat, token_table, pos_table)


def kernel(x, token_table, pos_table):
    out = _embed(x.astype(jnp.int32).reshape(-1), token_table, pos_table)
    return out.reshape(BATCH, MAXLEN, EMBED)
